# baseline (speedup 1.0000x reference)
import functools

import jax
import jax.numpy as jnp
from jax import lax
from jax.experimental import pallas as pl
from jax.experimental.pallas import tpu as pltpu
from jax.experimental.pallas import tpu_sc as plsc

_NC = 2
_NS = 16
_NW = _NC * _NS


def _sc_gather(table, idx, B, D):
    b_per_w = B // _NW
    mesh = plsc.VectorSubcoreMesh(core_axis_name="c", subcore_axis_name="s")

    @functools.partial(
        pl.kernel,
        out_type=jax.ShapeDtypeStruct((B, D), jnp.float32),
        mesh=mesh,
        scratch_types=[
            pltpu.VMEM((b_per_w,), jnp.int32),
            pltpu.VMEM((b_per_w, D), jnp.float32),
            pltpu.SemaphoreType.DMA,
        ],
    )
    def gather_kernel(table_hbm, idx_hbm, out_hbm, idx_v, rows_v, sem):
        wid = lax.axis_index("s") * _NC + lax.axis_index("c")
        base = wid * b_per_w
        pltpu.sync_copy(idx_hbm.at[pl.ds(base, b_per_w)], idx_v)
        pltpu.async_copy(table_hbm.at[idx_v], rows_v, sem).wait()
        pltpu.sync_copy(rows_v, out_hbm.at[pl.ds(base, b_per_w)])

    return gather_kernel(table, idx)


def _tc_multiply_chunk(g, y, buf, row_off, B, H, D, blk):
    nrows = g.shape[0]
    off = row_off // blk

    def mul_body(*refs):
        g_ref, y_ref, o_ref = refs[0], refs[1], refs[-1]
        g_blk = g_ref[...]
        y_blk = y_ref[...]
        o_ref[...] = g_blk[:, None, :] * y_blk[None, :, :]

    in_specs = [
        pl.BlockSpec((blk, D), lambda i: (i, 0)),
        pl.BlockSpec((H, D), lambda i: (0, 0)),
    ]
    args = [g, y]
    aliases = {}
    if buf is not None:
        in_specs.append(pl.BlockSpec(memory_space=pl.ANY))
        args.append(buf)
        aliases = {2: 0}
    return pl.pallas_call(
        mul_body,
        grid=(nrows // blk,),
        in_specs=in_specs,
        out_specs=pl.BlockSpec((blk, H, D), lambda i: (i + off, 0, 0)),
        out_shape=jax.ShapeDtypeStruct((B, H, D), jnp.float32),
        input_output_aliases=aliases,
    )(*args)


@jax.jit
def kernel(x, y, pos_id):
    V, D = x.shape[2], x.shape[3]
    H = y.shape[1]
    B = pos_id.shape[0]
    table = x.reshape(V, D)
    idx = pos_id.reshape(B)
    y2 = y.reshape(H, D)
    nchunk = 2
    C = B // nchunk
    g_chunks = [
        _sc_gather(table, lax.slice(idx, [c * C], [(c + 1) * C]), C, D)
        for c in range(nchunk)
    ]
    buf = None
    for c in range(nchunk):
        buf = _tc_multiply_chunk(g_chunks[c], y2, buf, c * C, B, H, D, blk=512)
    return buf.reshape(B, H, 1, D)


# speedup vs baseline: 1.0515x; 1.0515x over previous
import functools

import jax
import jax.numpy as jnp
from jax import lax
from jax.experimental import pallas as pl
from jax.experimental.pallas import tpu as pltpu
from jax.experimental.pallas import tpu_sc as plsc

_NC = 2
_NS = 16
_NW = _NC * _NS


def _sc_gather(table, idx, B, D):
    b_per_w = B // _NW
    mesh = plsc.VectorSubcoreMesh(core_axis_name="c", subcore_axis_name="s")

    half = b_per_w // 2

    @functools.partial(
        pl.kernel,
        out_type=jax.ShapeDtypeStruct((B, D), jnp.float32),
        mesh=mesh,
        scratch_types=[
            pltpu.VMEM((half,), jnp.int32),
            pltpu.VMEM((half,), jnp.int32),
            pltpu.VMEM((half, D), jnp.float32),
            pltpu.VMEM((half, D), jnp.float32),
            pltpu.SemaphoreType.DMA,
            pltpu.SemaphoreType.DMA,
            pltpu.SemaphoreType.DMA,
            pltpu.SemaphoreType.DMA,
        ],
    )
    def gather_kernel(
        table_hbm, idx_hbm, out_hbm,
        idx_v0, idx_v1, rows_v0, rows_v1, sem0, sem1, sem2, sem3,
    ):
        wid = lax.axis_index("s") * _NC + lax.axis_index("c")
        base = wid * b_per_w
        pltpu.sync_copy(idx_hbm.at[pl.ds(base, half)], idx_v0)
        g0 = pltpu.async_copy(table_hbm.at[idx_v0], rows_v0, sem0)
        pltpu.sync_copy(idx_hbm.at[pl.ds(base + half, half)], idx_v1)
        g1 = pltpu.async_copy(table_hbm.at[idx_v1], rows_v1, sem1)
        g0.wait()
        w0 = pltpu.async_copy(rows_v0, out_hbm.at[pl.ds(base, half)], sem2)
        g1.wait()
        w1 = pltpu.async_copy(rows_v1, out_hbm.at[pl.ds(base + half, half)], sem3)
        w0.wait()
        w1.wait()

    return gather_kernel(table, idx)


def _tc_multiply_chunk(g, y, buf, row_off, B, H, D, blk):
    nrows = g.shape[0]
    off = row_off // blk

    def mul_body(*refs):
        g_ref, y_ref, o_ref = refs[0], refs[1], refs[-1]
        g_blk = g_ref[...]
        y_blk = y_ref[...]
        o_ref[...] = g_blk[:, None, :] * y_blk[None, :, :]

    in_specs = [
        pl.BlockSpec((blk, D), lambda i: (i, 0)),
        pl.BlockSpec((H, D), lambda i: (0, 0)),
    ]
    args = [g, y]
    aliases = {}
    if buf is not None:
        in_specs.append(pl.BlockSpec(memory_space=pl.ANY))
        args.append(buf)
        aliases = {2: 0}
    return pl.pallas_call(
        mul_body,
        grid=(nrows // blk,),
        in_specs=in_specs,
        out_specs=pl.BlockSpec((blk, H, D), lambda i: (i + off, 0, 0)),
        out_shape=jax.ShapeDtypeStruct((B, H, D), jnp.float32),
        input_output_aliases=aliases,
    )(*args)


@jax.jit
def kernel(x, y, pos_id):
    V, D = x.shape[2], x.shape[3]
    H = y.shape[1]
    B = pos_id.shape[0]
    table = x.reshape(V, D)
    idx = pos_id.reshape(B)
    y2 = y.reshape(H, D)
    g = _sc_gather(table, idx, B, D)
    buf = _tc_multiply_chunk(g, y2, None, 0, B, H, D, blk=512)
    return buf.reshape(B, H, 1, D)
